# baseline (speedup 1.0000x reference)
import functools

import jax
import jax.numpy as jnp
from jax import lax
from jax.experimental import pallas as pl
from jax.experimental.pallas import tpu as pltpu
from jax.experimental.pallas import tpu_sc as plsc

EMBED_DIM = 64
HIDDEN = 128

NC = 2
NS = 16
NW = NC * NS
CHUNK = 128
K = 4
GROUP = K * CHUNK
BM = 2 * GROUP
WROWS = GROUP // 2


def _gather_body(table_hbm, idx_hbm, pk_hbm, idx_v, rows_v, pk_v,
                 sem0, sem1, *, chunks_per_w):
    wid = lax.axis_index("s") * NC + lax.axis_index("c")
    pltpu.sync_copy(idx_hbm.at[wid], idx_v)
    n_groups = chunks_per_w // K
    gbase = wid * n_groups
    sems = (sem0, sem1)

    def fire(g, buf):
        for b in range(K):
            pltpu.async_copy(
                table_hbm.at[idx_v.at[g * K + b]],
                rows_v.at[buf, pl.ds(b * CHUNK, CHUNK)],
                sems[buf])

    def drain(buf):
        for b in range(K):
            pltpu.make_async_copy(
                table_hbm.at[idx_v.at[0]],
                rows_v.at[buf, pl.ds(b * CHUNK, CHUNK)],
                sems[buf]).wait()

    def pack_write(g, buf):
        def rbody(r, carry):
            for k in range(EMBED_DIM // 16):
                wa = rows_v[buf, 2 * r, pl.ds(k * 16, 16)]
                wb = rows_v[buf, 2 * r + 1, pl.ds(k * 16, 16)]
                word = jnp.bitwise_or(
                    lax.shift_right_logical(wa, jnp.int32(16)),
                    jnp.bitwise_and(wb, jnp.int32(-65536)))
                pk_v[r, pl.ds(k * 16, 16)] = word
            return carry

        lax.fori_loop(0, WROWS, rbody, 0)
        gg = gbase + g
        dst = pk_hbm.at[pl.ds((gg // 2) * WROWS, WROWS),
                        pl.ds((gg % 2) * EMBED_DIM, EMBED_DIM)]
        pltpu.sync_copy(pk_v, dst)

    fire(0, 0)

    def group2(gg, carry):
        g0 = 2 * gg
        fire(g0 + 1, 1)
        drain(0)
        pack_write(g0, 0)

        @pl.when(g0 + 2 < n_groups)
        def _():
            fire(g0 + 2, 0)

        drain(1)
        pack_write(g0 + 1, 1)
        return carry

    lax.fori_loop(0, n_groups // 2, group2, 0)


def _sc_gather(table, idx3d, m):
    chunks_per_w = idx3d.shape[1]
    mesh = plsc.VectorSubcoreMesh(core_axis_name="c", subcore_axis_name="s")
    body = functools.partial(_gather_body, chunks_per_w=chunks_per_w)
    return pl.kernel(
        body,
        mesh=mesh,
        compiler_params=pltpu.CompilerParams(use_tc_tiling_on_sc=False),
        out_type=jax.ShapeDtypeStruct((m // 4, 2 * EMBED_DIM), jnp.int32),
        scratch_types=[
            pltpu.VMEM((chunks_per_w, CHUNK), jnp.int32),
            pltpu.VMEM((2, GROUP, EMBED_DIM), jnp.int32),
            pltpu.VMEM((WROWS, EMBED_DIM), jnp.int32),
            pltpu.SemaphoreType.DMA,
            pltpu.SemaphoreType.DMA,
        ],
    )(table, idx3d)


TC_BM = 16384


def _mm_body(p_ref, w_ref, o_ref):
    w = w_ref[...]
    e = pltpu.bitcast(p_ref[...], jnp.bfloat16)
    for p in range(TC_BM // BM):
        ep = e[p * GROUP:(p + 1) * GROUP, :]
        o_ref[pl.ds(p * BM, GROUP), :] = jnp.dot(
            ep[:, :EMBED_DIM], w, preferred_element_type=jnp.float32)
        o_ref[pl.ds(p * BM + GROUP, GROUP), :] = jnp.dot(
            ep[:, EMBED_DIM:], w, preferred_element_type=jnp.float32)


def _tc_project(pk, wt, m):
    return pl.pallas_call(
        _mm_body,
        grid=(m // TC_BM,),
        in_specs=[
            pl.BlockSpec((TC_BM // 4, 2 * EMBED_DIM), lambda i: (i, 0)),
            pl.BlockSpec((EMBED_DIM, HIDDEN), lambda i: (0, 0)),
        ],
        out_specs=pl.BlockSpec((TC_BM, HIDDEN), lambda i: (i, 0)),
        out_shape=jax.ShapeDtypeStruct((m, HIDDEN), jnp.float32),
    )(pk, wt)


def kernel(x, table, W):
    b, l = x.shape
    m = b * l
    cpw = m // (NW * CHUNK)
    idx3d = x.reshape(NW, cpw, CHUNK)
    table_i32 = lax.bitcast_convert_type(table, jnp.int32)
    pk = _sc_gather(table_i32, idx3d, m)
    wt = W.T.astype(jnp.bfloat16)
    out = _tc_project(pk, wt, m)
    return out.reshape(b, l, HIDDEN)


# speedup vs baseline: 1.1787x; 1.1787x over previous
import functools

import jax
import jax.numpy as jnp
from jax import lax
from jax.experimental import pallas as pl
from jax.experimental.pallas import tpu as pltpu
from jax.experimental.pallas import tpu_sc as plsc

EMBED_DIM = 64
HIDDEN = 128

NC = 2
NS = 16
NW = NC * NS
CHUNK = 128
K = 4
GROUP = K * CHUNK
BM = 2 * GROUP
WROWS = GROUP // 2


def _gather_body(table_hbm, idx_hbm, pk_hbm, idx_v, rows_v, pk_v,
                 sem0, sem1, *, chunks_per_w):
    wid = lax.axis_index("s") * NC + lax.axis_index("c")
    pltpu.sync_copy(idx_hbm.at[wid], idx_v)
    n_groups = chunks_per_w // K
    gbase = wid * n_groups
    sems = (sem0, sem1)

    def fire(g, buf):
        for b in range(K):
            pltpu.async_copy(
                table_hbm.at[idx_v.at[g * K + b]],
                rows_v.at[buf, pl.ds(b * CHUNK, CHUNK)],
                sems[buf])

    def drain(buf):
        for b in range(K):
            pltpu.make_async_copy(
                table_hbm.at[idx_v.at[0]],
                rows_v.at[buf, pl.ds(b * CHUNK, CHUNK)],
                sems[buf]).wait()

    def pack_write(g, buf):
        @plsc.parallel_loop(0, WROWS, unroll=8)
        def rbody(r):
            for k in range(EMBED_DIM // 16):
                wa = rows_v[buf, 2 * r, pl.ds(k * 16, 16)]
                wb = rows_v[buf, 2 * r + 1, pl.ds(k * 16, 16)]
                word = jnp.bitwise_or(
                    lax.shift_right_logical(wa, jnp.int32(16)),
                    jnp.bitwise_and(wb, jnp.int32(-65536)))
                pk_v[r, pl.ds(k * 16, 16)] = word
        gg = gbase + g
        dst = pk_hbm.at[pl.ds((gg // 2) * WROWS, WROWS),
                        pl.ds((gg % 2) * EMBED_DIM, EMBED_DIM)]
        pltpu.sync_copy(pk_v, dst)

    fire(0, 0)

    def group2(gg, carry):
        g0 = 2 * gg
        fire(g0 + 1, 1)
        drain(0)
        pack_write(g0, 0)

        @pl.when(g0 + 2 < n_groups)
        def _():
            fire(g0 + 2, 0)

        drain(1)
        pack_write(g0 + 1, 1)
        return carry

    lax.fori_loop(0, n_groups // 2, group2, 0)


def _sc_gather(table, idx3d, m):
    chunks_per_w = idx3d.shape[1]
    mesh = plsc.VectorSubcoreMesh(core_axis_name="c", subcore_axis_name="s")
    body = functools.partial(_gather_body, chunks_per_w=chunks_per_w)
    return pl.kernel(
        body,
        mesh=mesh,
        compiler_params=pltpu.CompilerParams(use_tc_tiling_on_sc=False),
        out_type=jax.ShapeDtypeStruct((m // 4, 2 * EMBED_DIM), jnp.int32),
        scratch_types=[
            pltpu.VMEM((chunks_per_w, CHUNK), jnp.int32),
            pltpu.VMEM((2, GROUP, EMBED_DIM), jnp.int32),
            pltpu.VMEM((WROWS, EMBED_DIM), jnp.int32),
            pltpu.SemaphoreType.DMA,
            pltpu.SemaphoreType.DMA,
        ],
    )(table, idx3d)


TC_BM = 16384


def _mm_body(p_ref, w_ref, o_ref):
    w = w_ref[...]
    e = pltpu.bitcast(p_ref[...], jnp.bfloat16)
    for p in range(TC_BM // BM):
        ep = e[p * GROUP:(p + 1) * GROUP, :]
        o_ref[pl.ds(p * BM, GROUP), :] = jnp.dot(
            ep[:, :EMBED_DIM], w, preferred_element_type=jnp.float32)
        o_ref[pl.ds(p * BM + GROUP, GROUP), :] = jnp.dot(
            ep[:, EMBED_DIM:], w, preferred_element_type=jnp.float32)


def _tc_project(pk, wt, m):
    return pl.pallas_call(
        _mm_body,
        grid=(m // TC_BM,),
        in_specs=[
            pl.BlockSpec((TC_BM // 4, 2 * EMBED_DIM), lambda i: (i, 0)),
            pl.BlockSpec((EMBED_DIM, HIDDEN), lambda i: (0, 0)),
        ],
        out_specs=pl.BlockSpec((TC_BM, HIDDEN), lambda i: (i, 0)),
        out_shape=jax.ShapeDtypeStruct((m, HIDDEN), jnp.float32),
    )(pk, wt)


def kernel(x, table, W):
    b, l = x.shape
    m = b * l
    cpw = m // (NW * CHUNK)
    idx3d = x.reshape(NW, cpw, CHUNK)
    table_i32 = lax.bitcast_convert_type(table, jnp.int32)
    pk = _sc_gather(table_i32, idx3d, m)
    wt = W.T.astype(jnp.bfloat16)
    out = _tc_project(pk, wt, m)
    return out.reshape(b, l, HIDDEN)


# speedup vs baseline: 1.2858x; 1.0908x over previous
import functools

import jax
import jax.numpy as jnp
from jax import lax
from jax.experimental import pallas as pl
from jax.experimental.pallas import tpu as pltpu
from jax.experimental.pallas import tpu_sc as plsc

EMBED_DIM = 64
HIDDEN = 128

NC = 2
NS = 16
NW = NC * NS
CHUNK = 128
K = 8


GROUP = K * CHUNK
BM = 2 * GROUP


def _gather_body(table_hbm, idx_hbm, emb2_hbm, idx_v, rows_v, gsem,
                 *, chunks_per_w):
    wid = lax.axis_index("s") * NC + lax.axis_index("c")
    pltpu.sync_copy(idx_hbm.at[wid], idx_v)
    n_groups = chunks_per_w // K
    gbase = wid * n_groups

    def group(g, carry):
        copies = []
        for b in range(K):
            j = g * K + b
            copies.append(pltpu.async_copy(
                table_hbm.at[idx_v.at[j]],
                rows_v.at[pl.ds(b * CHUNK, CHUNK)],
                gsem))
        for c in copies:
            c.wait()
        gg = gbase + g
        dst = emb2_hbm.at[pl.ds((gg // 2) * GROUP, GROUP),
                          pl.ds((gg % 2) * EMBED_DIM, EMBED_DIM)]
        pltpu.sync_copy(rows_v, dst)
        return carry

    lax.fori_loop(0, n_groups, group, 0)


def _sc_gather(table, idx3d, m):
    chunks_per_w = idx3d.shape[1]
    mesh = plsc.VectorSubcoreMesh(core_axis_name="c", subcore_axis_name="s")
    body = functools.partial(_gather_body, chunks_per_w=chunks_per_w)
    return pl.kernel(
        body,
        mesh=mesh,
        compiler_params=pltpu.CompilerParams(use_tc_tiling_on_sc=False),
        out_type=jax.ShapeDtypeStruct((m // 2, 2 * EMBED_DIM), jnp.float32),
        scratch_types=[
            pltpu.VMEM((chunks_per_w, CHUNK), jnp.int32),
            pltpu.VMEM((GROUP, EMBED_DIM), jnp.float32),
            pltpu.SemaphoreType.DMA,
        ],
    )(table, idx3d)


TC_BM = 16384


def _mm_body(e_ref, w_ref, o_ref):
    w = w_ref[...]
    for p in range(TC_BM // BM):
        e = e_ref[pl.ds(p * GROUP, GROUP), :]
        o_ref[pl.ds(p * BM, GROUP), :] = jnp.dot(
            e[:, :EMBED_DIM], w, preferred_element_type=jnp.float32)
        o_ref[pl.ds(p * BM + GROUP, GROUP), :] = jnp.dot(
            e[:, EMBED_DIM:], w, preferred_element_type=jnp.float32)


def _mm_body_alias(e_ref, w_ref, oprev_ref, o_ref):
    del oprev_ref
    _mm_body(e_ref, w_ref, o_ref)


def _tc_project_chunk(emb2_c, wt, out_prev, c, m_c, m):
    nblk = m_c // TC_BM
    e_spec = pl.BlockSpec((TC_BM // 2, 2 * EMBED_DIM), lambda i: (i, 0))
    w_spec = pl.BlockSpec((EMBED_DIM, HIDDEN), lambda i: (0, 0))
    o_spec = pl.BlockSpec((TC_BM, HIDDEN),
                          lambda i, c=c, nblk=nblk: (c * nblk + i, 0))
    out_shape = jax.ShapeDtypeStruct((m, HIDDEN), jnp.float32)
    if out_prev is None:
        return pl.pallas_call(
            _mm_body,
            grid=(nblk,),
            in_specs=[e_spec, w_spec],
            out_specs=o_spec,
            out_shape=out_shape,
        )(emb2_c, wt)
    return pl.pallas_call(
        _mm_body_alias,
        grid=(nblk,),
        in_specs=[e_spec, w_spec,
                  pl.BlockSpec(memory_space=pltpu.MemorySpace.HBM)],
        out_specs=o_spec,
        out_shape=out_shape,
        input_output_aliases={2: 0},
    )(emb2_c, wt, out_prev)


N_CHUNKS = 5


def kernel(x, table, W):
    b, l = x.shape
    m = b * l
    m_c = m // N_CHUNKS
    cpw = m_c // (NW * CHUNK)
    x_flat = x.reshape(-1)
    wt = W.T
    out = None
    for c in range(N_CHUNKS):
        idx3d = x_flat[c * m_c:(c + 1) * m_c].reshape(NW, cpw, CHUNK)
        emb2_c = _sc_gather(table, idx3d, m_c)
        out = _tc_project_chunk(emb2_c, wt, out, c, m_c, m)
    return out.reshape(b, l, HIDDEN)
